# baseline (speedup 1.0000x reference)
import functools

import jax
import jax.numpy as jnp
from jax.experimental import pallas as pl
from jax.experimental.pallas import tpu as pltpu

_LOG2E = 1.4426950408889634


def _attn_body(n_blocks, k_blocks, bs, dh, scale,
               bi_ref, q_ref, k_ref, v_ref, o_ref, kb_ref, vb_ref):
    kb_ref[...] = k_ref[0].astype(jnp.bfloat16)
    vb_ref[...] = v_ref[0].astype(jnp.bfloat16)

    for n in range(n_blocks):
        q = (q_ref[0, pl.ds(n * bs, bs), :] * (scale * _LOG2E)
             ).astype(jnp.bfloat16)
        kg = []
        vg = []
        for j in range(k_blocks):
            idx = bi_ref[n * k_blocks + j]
            kg.append(kb_ref[pl.ds(idx * bs, bs), :])
            vg.append(vb_ref[pl.ds(idx * bs, bs), :])
        kg = jnp.concatenate(kg, axis=0)
        vg = jnp.concatenate(vg, axis=0)
        s = jax.lax.dot_general(q, kg, (((1,), (1,)), ((), ())),
                                preferred_element_type=jnp.float32)
        e32 = jnp.exp2(s)
        e = e32.astype(jnp.bfloat16)
        u = jax.lax.dot_general(e, vg, (((1,), (0,)), ((), ())),
                                preferred_element_type=jnp.float32)
        d = jnp.sum(e32, axis=1, keepdims=True)
        o_ref[0, pl.ds(n * bs, bs), :] = u / d


def kernel(query, key, value, block_index):
    B, H, S, Dh = query.shape
    n_blocks, k_blocks = block_index.shape
    bs = S // n_blocks
    BH = B * H
    scale = 1.0 / float(Dh) ** 0.5

    q3 = query.reshape(BH, S, Dh)
    k3 = key.reshape(BH, S, Dh)
    v3 = value.reshape(BH, S, Dh)
    bi = block_index.reshape(-1).astype(jnp.int32)

    body = functools.partial(_attn_body, n_blocks, k_blocks, bs, Dh, scale)
    out = pl.pallas_call(
        body,
        grid_spec=pltpu.PrefetchScalarGridSpec(
            num_scalar_prefetch=1,
            grid=(BH,),
            in_specs=[
                pl.BlockSpec((1, S, Dh), lambda bh, bi_ref: (bh, 0, 0)),
                pl.BlockSpec((1, S, Dh), lambda bh, bi_ref: (bh, 0, 0)),
                pl.BlockSpec((1, S, Dh), lambda bh, bi_ref: (bh, 0, 0)),
            ],
            out_specs=pl.BlockSpec((1, S, Dh), lambda bh, bi_ref: (bh, 0, 0)),
            scratch_shapes=[
                pltpu.VMEM((S, Dh), jnp.bfloat16),
                pltpu.VMEM((S, Dh), jnp.bfloat16),
            ],
        ),
        out_shape=jax.ShapeDtypeStruct((BH, S, Dh), jnp.float32),
    )(bi, q3, k3, v3)
    return out.reshape(B, H, S, Dh)
